# baseline (speedup 1.0000x reference)
import jax
import jax.numpy as jnp
from jax import lax
from jax.experimental import pallas as pl
from jax.experimental.pallas import tpu as pltpu
from jax.experimental.pallas import tpu_sc as plsc

B = 16384
NCAT = 20
NNUM = 6
NCOL = NCAT + NNUM
VCAT = 100000
VNUM = 1000
D = 32

NC = 2
NS = 16
NW = NC * NS
BPW = B // NW
NB = 64
NBLK = BPW // NB
CHUNK = NCOL * NB
UNROLL = 4


def _sc_body(cat_tab, num_tab, idx_flat, bias_flat, out,
             idx_v, rows0, rows1, blk_v, bias_v,
             sem_g0, sem_g1, sem_o):
    wid = lax.axis_index("s") * NC + lax.axis_index("c")
    pltpu.sync_copy(bias_flat, bias_v)

    def fire_gather(c):
        tab = cat_tab if c < NCAT else num_tab
        rv = rows0 if c % 2 == 0 else rows1
        sem = sem_g0 if c % 2 == 0 else sem_g1
        return pltpu.async_copy(tab.at[idx_v.at[pl.ds(c * NB, NB)]], rv, sem)

    def do_block(blk, _):
        g_id = wid * NBLK + blk
        b0 = wid * BPW + blk * NB
        pltpu.sync_copy(idx_flat.at[pl.ds(g_id * CHUNK, CHUNK)], idx_v)
        pending = fire_gather(0)
        @pl.when(blk > 0)
        def _drain():
            pltpu.make_async_copy(blk_v, out.at[pl.ds(0, NB)], sem_o).wait()

        for c in range(NCOL):
            nxt = fire_gather(c + 1) if c + 1 < NCOL else None
            pending.wait()
            rv = rows0 if c % 2 == 0 else rows1
            b_lo = bias_v[pl.ds(c * D, 16)]
            b_hi = bias_v[pl.ds(c * D + 16, 16)]

            def add_pass(iv, carry, rv=rv, b_lo=b_lo, b_hi=b_hi, c=c):
                for u in range(UNROLL):
                    i = iv * UNROLL + u
                    blk_v[i, c, pl.ds(0, 16)] = rv[i, pl.ds(0, 16)] + b_lo
                    blk_v[i, c, pl.ds(16, 16)] = rv[i, pl.ds(16, 16)] + b_hi
                return carry

            lax.fori_loop(0, NB // UNROLL, add_pass, None)
            pending = nxt
        pltpu.async_copy(blk_v, out.at[pl.ds(b0, NB)], sem_o)
        return _

    lax.fori_loop(0, NBLK, do_block, None)
    pltpu.make_async_copy(blk_v, out.at[pl.ds(0, NB)], sem_o).wait()


@jax.jit
def kernel(cat_idx, num_idx, cat_tables, cat_bias, num_tables, num_bias):
    idx_cat = cat_idx + jnp.arange(NCAT, dtype=jnp.int32)[None, :] * VCAT
    idx_num = num_idx + jnp.arange(NNUM, dtype=jnp.int32)[None, :] * VNUM
    idx_all = jnp.concatenate([idx_cat, idx_num], axis=1)
    idx_flat = idx_all.reshape(B // NB, NB, NCOL).transpose(0, 2, 1).reshape(-1)

    cat_tab = cat_tables.reshape(NCAT * VCAT, D)
    num_tab = num_tables.reshape(NNUM * VNUM, D)
    bias_flat = jnp.concatenate([cat_bias, num_bias], axis=0).reshape(-1)

    mesh = plsc.VectorSubcoreMesh(core_axis_name="c", subcore_axis_name="s")
    return pl.kernel(
        _sc_body,
        mesh=mesh,
        compiler_params=pltpu.CompilerParams(use_tc_tiling_on_sc=False,
                                             needs_layout_passes=False),
        out_type=jax.ShapeDtypeStruct((B, NCOL, D), jnp.float32),
        scratch_types=[
            pltpu.VMEM((CHUNK,), jnp.int32),
            pltpu.VMEM((NB, D), jnp.float32),
            pltpu.VMEM((NB, D), jnp.float32),
            pltpu.VMEM((NB, NCOL, D), jnp.float32),
            pltpu.VMEM((NCOL * D,), jnp.float32),
            pltpu.SemaphoreType.DMA,
            pltpu.SemaphoreType.DMA,
            pltpu.SemaphoreType.DMA,
        ],
    )(cat_tab, num_tab, idx_flat, bias_flat)


# speedup vs baseline: 1.0007x; 1.0007x over previous
import jax
import jax.numpy as jnp
from jax import lax
from jax.experimental import pallas as pl
from jax.experimental.pallas import tpu as pltpu
from jax.experimental.pallas import tpu_sc as plsc

B = 16384
NCAT = 20
NNUM = 6
NCOL = NCAT + NNUM
VCAT = 100000
VNUM = 1000
D = 32

NC = 2
NS = 16
NW = NC * NS
BPW = B // NW
NB = 64
NBLK = BPW // NB
CHUNK = NCOL * NB
UNROLL = 4


def _sc_body(cat_tab, num_tab, idx_flat, bias_flat, out,
             idx_v, rows0, rows1, blk_v, bias_v,
             sem_g0, sem_g1, sem_o):
    wid = lax.axis_index("s") * NC + lax.axis_index("c")
    pltpu.sync_copy(bias_flat, bias_v)

    def fire_gather(c):
        tab = cat_tab.at[c] if c < NCAT else num_tab.at[c - NCAT]
        rv = rows0 if c % 2 == 0 else rows1
        sem = sem_g0 if c % 2 == 0 else sem_g1
        return pltpu.async_copy(tab.at[idx_v.at[pl.ds(c * NB, NB)]], rv, sem)

    def do_block(blk, _):
        g_id = wid * NBLK + blk
        b0 = wid * BPW + blk * NB
        pltpu.sync_copy(idx_flat.at[pl.ds(g_id * CHUNK, CHUNK)], idx_v)
        pending = fire_gather(0)
        @pl.when(blk > 0)
        def _drain():
            pltpu.make_async_copy(blk_v, out.at[pl.ds(0, NB)], sem_o).wait()

        for c in range(NCOL):
            nxt = fire_gather(c + 1) if c + 1 < NCOL else None
            pending.wait()
            rv = rows0 if c % 2 == 0 else rows1
            b_lo = bias_v[pl.ds(c * D, 16)]
            b_hi = bias_v[pl.ds(c * D + 16, 16)]

            def add_pass(iv, carry, rv=rv, b_lo=b_lo, b_hi=b_hi, c=c):
                for u in range(UNROLL):
                    i = iv * UNROLL + u
                    blk_v[i, c, pl.ds(0, 16)] = rv[i, pl.ds(0, 16)] + b_lo
                    blk_v[i, c, pl.ds(16, 16)] = rv[i, pl.ds(16, 16)] + b_hi
                return carry

            lax.fori_loop(0, NB // UNROLL, add_pass, None)
            pending = nxt
        pltpu.async_copy(blk_v, out.at[pl.ds(b0, NB)], sem_o)
        return _

    lax.fori_loop(0, NBLK, do_block, None)
    pltpu.make_async_copy(blk_v, out.at[pl.ds(0, NB)], sem_o).wait()


@jax.jit
def kernel(cat_idx, num_idx, cat_tables, cat_bias, num_tables, num_bias):
    idx_all = jnp.concatenate([cat_idx, num_idx], axis=1)
    idx_flat = idx_all.reshape(B // NB, NB, NCOL).transpose(0, 2, 1).reshape(-1)
    bias_flat = jnp.concatenate([cat_bias, num_bias], axis=0).reshape(-1)

    mesh = plsc.VectorSubcoreMesh(core_axis_name="c", subcore_axis_name="s")
    return pl.kernel(
        _sc_body,
        mesh=mesh,
        compiler_params=pltpu.CompilerParams(use_tc_tiling_on_sc=False,
                                             needs_layout_passes=False),
        out_type=jax.ShapeDtypeStruct((B, NCOL, D), jnp.float32),
        scratch_types=[
            pltpu.VMEM((CHUNK,), jnp.int32),
            pltpu.VMEM((NB, D), jnp.float32),
            pltpu.VMEM((NB, D), jnp.float32),
            pltpu.VMEM((NB, NCOL, D), jnp.float32),
            pltpu.VMEM((NCOL * D,), jnp.float32),
            pltpu.SemaphoreType.DMA,
            pltpu.SemaphoreType.DMA,
            pltpu.SemaphoreType.DMA,
        ],
    )(cat_tables, num_tables, idx_flat, bias_flat)


# speedup vs baseline: 1.0752x; 1.0744x over previous
import jax
import jax.numpy as jnp
from jax import lax
from jax.experimental import pallas as pl
from jax.experimental.pallas import tpu as pltpu
from jax.experimental.pallas import tpu_sc as plsc

B = 16384
NCAT = 20
NNUM = 6
NCOL = NCAT + NNUM
VCAT = 100000
VNUM = 1000
D = 32

NC = 2
NS = 16
NW = NC * NS
BPW = B // NW
NB = 128
NBLK = BPW // NB
CHUNK = NCOL * NB
UNROLL = 4
S_COL = (D // 8) * (B // NB) * 8 * NB
S_R = (B // NB) * 8 * NB
S_BLK = 8 * NB


def _sc_body(cat_tab, num_tab, idx_flat, bias_flat, out,
             idx_v, rows0, rows1, blk0, blk1, bias_v,
             sem_g0, sem_g1, sem_o0, sem_o1):
    wid = lax.axis_index("s") * NC + lax.axis_index("c")
    pltpu.sync_copy(bias_flat, bias_v)
    iota = lax.iota(jnp.int32, 16)
    scat_lo = iota * NB
    scat_hi = (iota + 16) * NB

    def fire_gather(c):
        tab = cat_tab.at[c] if c < NCAT else num_tab.at[c - NCAT]
        rv = rows0 if c % 2 == 0 else rows1
        sem = sem_g0 if c % 2 == 0 else sem_g1
        return pltpu.async_copy(tab.at[idx_v.at[pl.ds(c * NB, NB)]], rv, sem)

    def do_block(blk, _):
        g_id = wid * NBLK + blk
        pltpu.sync_copy(idx_flat.at[pl.ds(g_id * CHUNK, CHUNK)], idx_v)
        pending_o = {0: [], 1: []}
        pending = fire_gather(0)
        for c in range(NCOL):
            nxt = fire_gather(c + 1) if c + 1 < NCOL else None
            pending.wait()
            pending = nxt
            for cp in pending_o[c % 2]:
                cp.wait()
            pending_o[c % 2] = []
            rv = rows0 if c % 2 == 0 else rows1
            bv = blk0 if c % 2 == 0 else blk1
            b_lo = bias_v[pl.ds(c * D, 16)]
            b_hi = bias_v[pl.ds(c * D + 16, 16)]

            def add_pass(iv, carry, rv=rv, bv=bv, b_lo=b_lo, b_hi=b_hi):
                for u in range(UNROLL):
                    i = iv * UNROLL + u
                    plsc.store_scatter(bv, [scat_lo + i],
                                       rv[i, pl.ds(0, 16)] + b_lo)
                    plsc.store_scatter(bv, [scat_hi + i],
                                       rv[i, pl.ds(16, 16)] + b_hi)
                return carry

            lax.fori_loop(0, NB // UNROLL, add_pass, None)
            sem_o = sem_o0 if c % 2 == 0 else sem_o1
            base = c * S_COL + g_id * S_BLK
            for r in range(D // 8):
                pending_o[c % 2].append(pltpu.async_copy(
                    bv.at[pl.ds(r * S_BLK, S_BLK)],
                    out.at[pl.ds(base + r * S_R, S_BLK)], sem_o))
        for par in (0, 1):
            for cp in pending_o[par]:
                cp.wait()
        return _

    lax.fori_loop(0, NBLK, do_block, None)


@jax.jit
def kernel(cat_idx, num_idx, cat_tables, cat_bias, num_tables, num_bias):
    idx_all = jnp.concatenate([cat_idx, num_idx], axis=1)
    idx_flat = idx_all.reshape(B // NB, NB, NCOL).transpose(0, 2, 1).reshape(-1)
    bias_flat = jnp.concatenate([cat_bias, num_bias], axis=0).reshape(-1)

    mesh = plsc.VectorSubcoreMesh(core_axis_name="c", subcore_axis_name="s")
    out = pl.kernel(
        _sc_body,
        mesh=mesh,
        compiler_params=pltpu.CompilerParams(use_tc_tiling_on_sc=False,
                                             needs_layout_passes=False),
        out_type=jax.ShapeDtypeStruct((B * NCOL * D,), jnp.float32),
        scratch_types=[
            pltpu.VMEM((CHUNK,), jnp.int32),
            pltpu.VMEM((NB, D), jnp.float32),
            pltpu.VMEM((NB, D), jnp.float32),
            pltpu.VMEM((D * NB,), jnp.float32),
            pltpu.VMEM((D * NB,), jnp.float32),
            pltpu.VMEM((NCOL * D,), jnp.float32),
            pltpu.SemaphoreType.DMA,
            pltpu.SemaphoreType.DMA,
            pltpu.SemaphoreType.DMA,
            pltpu.SemaphoreType.DMA,
        ],
    )(cat_tables, num_tables, idx_flat, bias_flat)

    x = out.reshape(NCOL, D // 8, B // NB, 8, NB)
    x = x.transpose(0, 1, 3, 2, 4)
    x = x.reshape(NCOL, D, B)
    return x.transpose(2, 0, 1)
